# Initial kernel scaffold; baseline (speedup 1.0000x reference)
import functools

import jax
import jax.numpy as jnp
from jax import lax
from jax.experimental import pallas as pl
from jax.experimental.pallas import tpu as pltpu
from jax.experimental.pallas import tpu_sc as plsc

MAXLEN = 200
D = 32
B = 4096
S = 200
TOTAL = B * S
NW = 32
ROWS_PER_W = TOTAL // NW
CHUNK = 128
NCHUNK = ROWS_PER_W // CHUNK
LANES = 16

_mesh = plsc.VectorSubcoreMesh(core_axis_name="c", subcore_axis_name="s")


@functools.partial(
    pl.kernel,
    mesh=_mesh,
    out_type=jax.ShapeDtypeStruct((TOTAL, D), jnp.float32),
    scratch_types=[
        pltpu.VMEM((NCHUNK, CHUNK), jnp.int32),
        pltpu.VMEM((2 * MAXLEN, D), jnp.float32),
        pltpu.VMEM((CHUNK, D), jnp.float32),
        pltpu.VMEM((CHUNK, D), jnp.float32),
        pltpu.SemaphoreType.DMA,
        pltpu.SemaphoreType.DMA,
    ],
)
def _embed(x_hbm, tok_hbm, pos_hbm, out_hbm, idx_v, pos2, rows0, rows1, sem0, sem1):
    wid = lax.axis_index("s") * 2 + lax.axis_index("c")
    base = wid * ROWS_PER_W

    pltpu.sync_copy(x_hbm.at[pl.ds(wid * NCHUNK, NCHUNK)], idx_v)
    pltpu.sync_copy(pos_hbm, pos2.at[pl.ds(0, MAXLEN)])
    pltpu.sync_copy(pos_hbm, pos2.at[pl.ds(MAXLEN, MAXLEN)])

    rows = (rows0, rows1)
    sems = (sem0, sem1)

    def gather_start(c, b):
        pltpu.async_copy(tok_hbm.at[idx_v.at[c]], rows[b], sems[b])

    def gather_wait(c, b):
        pltpu.make_async_copy(tok_hbm.at[idx_v.at[c]], rows[b], sems[b]).wait()

    gather_start(0, 0)
    gather_start(1, 1)

    def chunk_body(cc, carry):
        for b in range(2):
            c = 2 * cc + b
            gather_wait(c, b)
            p0 = lax.rem(c * CHUNK, MAXLEN)

            def row_body(i, carry2, _b=b, _p0=p0):
                for h in range(D // LANES):
                    sl = pl.ds(h * LANES, LANES)
                    rows[_b][i, sl] = rows[_b][i, sl] + pos2[_p0 + i, sl]
                return carry2

            lax.fori_loop(0, CHUNK, row_body, 0, unroll=4)
            pltpu.sync_copy(rows[b], out_hbm.at[pl.ds(base + c * CHUNK, CHUNK)])

            @pl.when(c + 2 < NCHUNK)
            def _(_c=c, _b=b):
                gather_start(_c + 2, _b)

        return carry

    lax.fori_loop(0, NCHUNK // 2, chunk_body, 0)


def kernel(x, token_table, pos_table):
    xf = x.astype(jnp.int32).reshape(TOTAL // CHUNK, CHUNK)
    out = _embed(xf, token_table, pos_table)
    return out.reshape(B, S, D)


# speedup vs baseline: 2.7245x; 2.7245x over previous
import functools

import jax
import jax.numpy as jnp
from jax import lax
from jax.experimental import pallas as pl
from jax.experimental.pallas import tpu as pltpu
from jax.experimental.pallas import tpu_sc as plsc

MAXLEN = 200
D = 32
B = 4096
S = 200
TOTAL = B * S
NW = 32
ROWS_PER_W = TOTAL // NW
CHUNK = 128
NCHUNK = ROWS_PER_W // CHUNK
LANES = 16

_mesh = plsc.VectorSubcoreMesh(core_axis_name="c", subcore_axis_name="s")


@functools.partial(
    pl.kernel,
    mesh=_mesh,
    out_type=jax.ShapeDtypeStruct((TOTAL, D), jnp.float32),
    compiler_params=pltpu.CompilerParams(use_tc_tiling_on_sc=False),
    scratch_types=[
        pltpu.VMEM((NCHUNK, CHUNK), jnp.int32),
        pltpu.VMEM((2 * MAXLEN, D), jnp.float32),
        pltpu.VMEM((CHUNK, D), jnp.float32),
        pltpu.VMEM((CHUNK, D), jnp.float32),
        pltpu.SemaphoreType.DMA,
        pltpu.SemaphoreType.DMA,
    ],
)
def _embed(x_hbm, tok_hbm, pos_hbm, out_hbm, idx_v, pos2, rows0, rows1, sem0, sem1):
    wid = lax.axis_index("s") * 2 + lax.axis_index("c")
    base = wid * ROWS_PER_W

    pltpu.sync_copy(x_hbm.at[pl.ds(wid * NCHUNK, NCHUNK)], idx_v)
    pltpu.sync_copy(pos_hbm, pos2.at[pl.ds(0, MAXLEN)])
    pltpu.sync_copy(pos_hbm, pos2.at[pl.ds(MAXLEN, MAXLEN)])

    rows = (rows0, rows1)
    sems = (sem0, sem1)

    def gather_start(c, b):
        pltpu.async_copy(tok_hbm.at[idx_v.at[c]], rows[b], sems[b])

    def gather_wait(c, b):
        pltpu.make_async_copy(tok_hbm.at[idx_v.at[c]], rows[b], sems[b]).wait()

    gather_start(0, 0)
    gather_start(1, 1)

    def chunk_body(cc, carry):
        for b in range(2):
            c = 2 * cc + b
            gather_wait(c, b)
            p0 = lax.rem(c * CHUNK, MAXLEN)

            def row_body(i, carry2, _b=b, _p0=p0):
                for h in range(D // LANES):
                    sl = pl.ds(h * LANES, LANES)
                    rows[_b][i, sl] = rows[_b][i, sl] + pos2[_p0 + i, sl]
                return carry2

            lax.fori_loop(0, CHUNK, row_body, 0, unroll=4)
            pltpu.sync_copy(rows[b], out_hbm.at[pl.ds(base + c * CHUNK, CHUNK)])

            @pl.when(c + 2 < NCHUNK)
            def _(_c=c, _b=b):
                gather_start(_c + 2, _b)

        return carry

    lax.fori_loop(0, NCHUNK // 2, chunk_body, 0)


def kernel(x, token_table, pos_table):
    xf = x.astype(jnp.int32).reshape(TOTAL // CHUNK, CHUNK)
    out = _embed(xf, token_table, pos_table)
    return out.reshape(B, S, D)


# speedup vs baseline: 3.0979x; 1.1371x over previous
import functools

import jax
import jax.numpy as jnp
from jax import lax
from jax.experimental import pallas as pl
from jax.experimental.pallas import tpu as pltpu
from jax.experimental.pallas import tpu_sc as plsc

MAXLEN = 200
D = 32
B = 4096
S = 200
NW = 32
BBLK = B // NW
LANES = 16

_mesh = plsc.VectorSubcoreMesh(core_axis_name="c", subcore_axis_name="s")


@functools.partial(
    pl.kernel,
    mesh=_mesh,
    out_type=jax.ShapeDtypeStruct((B, S, D), jnp.float32),
    compiler_params=pltpu.CompilerParams(use_tc_tiling_on_sc=False),
    scratch_types=[
        pltpu.VMEM((S, BBLK), jnp.int32),
        pltpu.VMEM((MAXLEN, D), jnp.float32),
        pltpu.VMEM((BBLK, D), jnp.float32),
        pltpu.VMEM((BBLK, D), jnp.float32),
        pltpu.SemaphoreType.DMA,
        pltpu.SemaphoreType.DMA,
    ],
)
def _embed(xt_hbm, tok_hbm, pos_hbm, out_hbm, idx_v, pos_v, rows0, rows1, sem0, sem1):
    wid = lax.axis_index("s") * 2 + lax.axis_index("c")
    b0 = wid * BBLK

    pltpu.sync_copy(xt_hbm.at[:, pl.ds(b0, BBLK)], idx_v)
    pltpu.sync_copy(pos_hbm, pos_v)

    rows = (rows0, rows1)
    sems = (sem0, sem1)

    def gather_start(s, b):
        pltpu.async_copy(tok_hbm.at[idx_v.at[s]], rows[b], sems[b])

    def gather_wait(s, b):
        pltpu.make_async_copy(tok_hbm.at[idx_v.at[s]], rows[b], sems[b]).wait()

    gather_start(0, 0)
    gather_start(1, 1)

    def chunk_body(ss, carry):
        for b in range(2):
            s = 2 * ss + b
            gather_wait(s, b)
            p0 = pos_v[s, pl.ds(0, LANES)]
            p1 = pos_v[s, pl.ds(LANES, LANES)]

            def row_body(i, carry2, _b=b, _p0=p0, _p1=p1):
                rows[_b][i, pl.ds(0, LANES)] = rows[_b][i, pl.ds(0, LANES)] + _p0
                rows[_b][i, pl.ds(LANES, LANES)] = rows[_b][i, pl.ds(LANES, LANES)] + _p1
                return carry2

            lax.fori_loop(0, BBLK, row_body, 0, unroll=4)
            pltpu.sync_copy(rows[b], out_hbm.at[pl.ds(b0, BBLK), s])

            @pl.when(s + 2 < S)
            def _(_s=s, _b=b):
                gather_start(_s + 2, _b)

        return carry

    lax.fori_loop(0, S // 2, chunk_body, 0)


def kernel(x, token_table, pos_table):
    xt = x.astype(jnp.int32).T
    return _embed(xt, token_table, pos_table)


# speedup vs baseline: 4.1176x; 1.3292x over previous
import functools

import jax
import jax.numpy as jnp
from jax import lax
from jax.experimental import pallas as pl
from jax.experimental.pallas import tpu as pltpu
from jax.experimental.pallas import tpu_sc as plsc

MAXLEN = 200
D = 32
B = 4096
S = 200
NW = 32
BBLK = B // NW
LANES = 16
TPITCH = BBLK + 1

_mesh = plsc.VectorSubcoreMesh(core_axis_name="c", subcore_axis_name="s")


@functools.partial(
    pl.kernel,
    mesh=_mesh,
    out_type=jax.ShapeDtypeStruct((S, D // 8, B // 128, 8, 128), jnp.float32),
    compiler_params=pltpu.CompilerParams(
        use_tc_tiling_on_sc=False, needs_layout_passes=False),
    scratch_types=[
        pltpu.VMEM((S, BBLK), jnp.int32),
        pltpu.VMEM((MAXLEN, D), jnp.float32),
        pltpu.VMEM((BBLK, D), jnp.float32),
        pltpu.VMEM((BBLK, D), jnp.float32),
        pltpu.VMEM((D // 8, 8, TPITCH), jnp.float32),
        pltpu.SemaphoreType.DMA,
        pltpu.SemaphoreType.DMA,
    ],
)
def _embed(xt_hbm, tok_hbm, pos_hbm, out_hbm, idx_v, pos_v, rows0, rows1, tbuf, sem0, sem1):
    wid = lax.axis_index("s") * 2 + lax.axis_index("c")
    b0 = wid * BBLK

    pltpu.sync_copy(xt_hbm.at[:, pl.ds(b0, BBLK)], idx_v)
    pltpu.sync_copy(pos_hbm, pos_v)

    rows = (rows0, rows1)
    sems = (sem0, sem1)

    lane = lax.iota(jnp.int32, 16)
    dl = lax.rem(lane, 8)
    dt0 = lax.div(lane, 8)
    dts = (dt0, dt0 + 2)
    dls = (dl, dl)

    def gather_start(s, b):
        pltpu.async_copy(tok_hbm.at[idx_v.at[s]], rows[b], sems[b])

    def gather_wait(s, b):
        pltpu.make_async_copy(tok_hbm.at[idx_v.at[s]], rows[b], sems[b]).wait()

    gather_start(0, 0)
    gather_start(1, 1)

    def chunk_body(ss, carry):
        for b in range(2):
            s = 2 * ss + b
            gather_wait(s, b)
            p0 = pos_v[s, pl.ds(0, LANES)]
            p1 = pos_v[s, pl.ds(LANES, LANES)]

            def row_body(i, carry2, _b=b, _p0=p0, _p1=p1):
                bi = jnp.full((LANES,), i, dtype=jnp.int32)
                v0 = rows[_b][i, pl.ds(0, LANES)] + _p0
                plsc.store_scatter(tbuf, [dts[0], dls[0], bi], v0)
                v1 = rows[_b][i, pl.ds(LANES, LANES)] + _p1
                plsc.store_scatter(tbuf, [dts[1], dls[1], bi], v1)
                return carry2

            lax.fori_loop(0, BBLK, row_body, 0, unroll=4)
            pltpu.sync_copy(tbuf.at[:, :, pl.ds(0, BBLK)], out_hbm.at[s, :, wid])

            @pl.when(s + 2 < S)
            def _(_s=s, _b=b):
                gather_start(_s + 2, _b)

        return carry

    lax.fori_loop(0, S // 2, chunk_body, 0)


def kernel(x, token_table, pos_table):
    xt = x.astype(jnp.int32).T
    out5 = _embed(xt, token_table, pos_table)
    return out5.transpose(2, 4, 0, 1, 3).reshape(B, S, D)


# speedup vs baseline: 4.1213x; 1.0009x over previous
import functools

import jax
import jax.numpy as jnp
from jax import lax
from jax.experimental import pallas as pl
from jax.experimental.pallas import tpu as pltpu
from jax.experimental.pallas import tpu_sc as plsc

MAXLEN = 200
D = 32
B = 4096
S = 200
NW = 32
BBLK = B // NW
LANES = 16
TPITCH = BBLK + 1

_mesh = plsc.VectorSubcoreMesh(core_axis_name="c", subcore_axis_name="s")


@functools.partial(
    pl.kernel,
    mesh=_mesh,
    out_type=jax.ShapeDtypeStruct((S, D // 8, B // 128, 8, 128), jnp.float32),
    compiler_params=pltpu.CompilerParams(
        use_tc_tiling_on_sc=False, needs_layout_passes=False,
        disable_bounds_checks=True),
    scratch_types=[
        pltpu.VMEM((S, BBLK), jnp.int32),
        pltpu.VMEM((MAXLEN, D), jnp.float32),
        pltpu.VMEM((BBLK, D), jnp.float32),
        pltpu.VMEM((BBLK, D), jnp.float32),
        pltpu.VMEM((D // 8, 8, TPITCH), jnp.float32),
        pltpu.SemaphoreType.DMA,
        pltpu.SemaphoreType.DMA,
    ],
)
def _embed(xt_hbm, tok_hbm, pos_hbm, out_hbm, idx_v, pos_v, rows0, rows1, tbuf, sem0, sem1):
    wid = lax.axis_index("s") * 2 + lax.axis_index("c")
    b0 = wid * BBLK

    pltpu.sync_copy(xt_hbm.at[:, pl.ds(b0, BBLK)], idx_v)
    pltpu.sync_copy(pos_hbm, pos_v)

    rows = (rows0, rows1)
    sems = (sem0, sem1)

    lane = lax.iota(jnp.int32, 16)
    dl = lax.rem(lane, 8)
    dt0 = lax.div(lane, 8)
    dts = (dt0, dt0 + 2)
    dls = (dl, dl)

    def gather_start(s, b):
        pltpu.async_copy(tok_hbm.at[idx_v.at[s]], rows[b], sems[b])

    def gather_wait(s, b):
        pltpu.make_async_copy(tok_hbm.at[idx_v.at[s]], rows[b], sems[b]).wait()

    gather_start(0, 0)
    gather_start(1, 1)

    def chunk_body(ss, carry):
        for b in range(2):
            s = 2 * ss + b
            gather_wait(s, b)
            p0 = pos_v[s, pl.ds(0, LANES)]
            p1 = pos_v[s, pl.ds(LANES, LANES)]

            def row_body(i, carry2, _b=b, _p0=p0, _p1=p1):
                bi = jnp.full((LANES,), i, dtype=jnp.int32)
                v0 = rows[_b][i, pl.ds(0, LANES)] + _p0
                plsc.store_scatter(tbuf, [dts[0], dls[0], bi], v0)
                v1 = rows[_b][i, pl.ds(LANES, LANES)] + _p1
                plsc.store_scatter(tbuf, [dts[1], dls[1], bi], v1)
                return carry2

            lax.fori_loop(0, BBLK, row_body, 0, unroll=4)
            pltpu.sync_copy(tbuf.at[:, :, pl.ds(0, BBLK)], out_hbm.at[s, :, wid])

            @pl.when(s + 2 < S)
            def _(_s=s, _b=b):
                gather_start(_s + 2, _b)

        return carry

    lax.fori_loop(0, S // 2, chunk_body, 0)


def kernel(x, token_table, pos_table):
    xt = x.astype(jnp.int32).T
    out5 = _embed(xt, token_table, pos_table)
    return out5.transpose(2, 4, 0, 1, 3).reshape(B, S, D)
